# Initial kernel scaffold; baseline (speedup 1.0000x reference)
import jax
import jax.numpy as jnp
from jax.experimental import pallas as pl


def kernel(inputs, tables, W1_0, b1_0, W2_0, b2_0, W1_1, b1_1, W2_1, b2_1, Wf, bf):
    raise NotImplementedError("write your pallas kernel here")


# speedup vs baseline: 1.1287x; 1.1287x over previous
import functools

import jax
import jax.numpy as jnp
from jax import lax
from jax.experimental import pallas as pl
from jax.experimental.pallas import tpu as pltpu
from jax.experimental.pallas import tpu_sc as plsc

B = 16384
F = 26
V = 100000
D = 16
DS = F * D
H = 256

NC = 2
NS = 16
NW = NC * NS
N = B * F
RW = N // NW
NCHUNK = 4
CH = RW // NCHUNK


def _gather_body(idx_hbm, tbl_hbm, out_hbm, idx_v, rows_v, sem):
    wid = lax.axis_index("s") * NC + lax.axis_index("c")
    base = wid * RW

    def chunk(i, carry):
        off = base + i * CH
        pltpu.sync_copy(idx_hbm.at[pl.ds(off, CH)], idx_v)
        pltpu.async_copy(tbl_hbm.at[idx_v], rows_v, sem).wait()
        pltpu.sync_copy(rows_v, out_hbm.at[pl.ds(off, CH)])
        return carry

    lax.fori_loop(0, NCHUNK, chunk, 0)


@jax.jit
def _sc_gather(flat_idx, tbl):
    mesh = plsc.VectorSubcoreMesh(core_axis_name="c", subcore_axis_name="s")
    return pl.kernel(
        _gather_body,
        out_type=jax.ShapeDtypeStruct((N, D), jnp.float32),
        mesh=mesh,
        scratch_types=[
            pltpu.VMEM((CH,), jnp.int32),
            pltpu.VMEM((CH, D), jnp.float32),
            pltpu.SemaphoreType.DMA,
        ],
        compiler_params=pltpu.CompilerParams(use_tc_tiling_on_sc=False),
    )(flat_idx, tbl)


def _mlp_body(x_ref, w10, b10, w20, b20, w11, b11, w21, b21, wf, bfr, out_ref):
    x = x_ref[...]
    h = jnp.maximum(jnp.dot(x, w10[...], preferred_element_type=jnp.float32) + b10[...], 0.0)
    x = jnp.maximum(x + jnp.dot(h, w20[...], preferred_element_type=jnp.float32) + b20[...], 0.0)
    h = jnp.maximum(jnp.dot(x, w11[...], preferred_element_type=jnp.float32) + b11[...], 0.0)
    x = jnp.maximum(x + jnp.dot(h, w21[...], preferred_element_type=jnp.float32) + b21[...], 0.0)
    z = jnp.dot(x, wf[...], preferred_element_type=jnp.float32) + bfr[...]
    out_ref[...] = jax.nn.sigmoid(z)


BB = 1024


def _mlp_call(embs, W1_0, b1_0, W2_0, b2_0, W1_1, b1_1, W2_1, b2_1, Wf, bf):
    full = lambda shape: pl.BlockSpec(shape, lambda i: (0,) * len(shape))
    return pl.pallas_call(
        _mlp_body,
        grid=(B // BB,),
        in_specs=[
            pl.BlockSpec((BB, DS), lambda i: (i, 0)),
            full((DS, H)), full((1, H)),
            full((H, DS)), full((1, DS)),
            full((DS, H)), full((1, H)),
            full((H, DS)), full((1, DS)),
            full((DS, 1)), full((1, 1)),
        ],
        out_specs=pl.BlockSpec((BB, 1), lambda i: (i, 0)),
        out_shape=jax.ShapeDtypeStruct((B, 1), jnp.float32),
    )(embs, W1_0, b1_0.reshape(1, H), W2_0, b2_0.reshape(1, DS),
      W1_1, b1_1.reshape(1, H), W2_1, b2_1.reshape(1, DS),
      Wf, bf.reshape(1, 1))


def kernel(inputs, tables, W1_0, b1_0, W2_0, b2_0, W1_1, b1_1, W2_1, b2_1, Wf, bf):
    flat_idx = (inputs + (jnp.arange(F, dtype=jnp.int32) * V)[None, :]).reshape(-1)
    tbl = tables.reshape(F * V, D)
    rows = _sc_gather(flat_idx, tbl)
    embs = rows.reshape(B, DS)
    return _mlp_call(embs, W1_0, b1_0, W2_0, b2_0, W1_1, b1_1, W2_1, b2_1, Wf, bf)
